# baseline (speedup 1.0000x reference)
import jax
import jax.numpy as jnp
from jax.experimental import pallas as pl

_TOP_K = 8
_BLOCK_ROWS = 512


def _router_block(x_ref, w_ref, idx_ref, tw_ref, probs_ref, logits_ref):
    logits = jnp.dot(x_ref[...], w_ref[...], preferred_element_type=jnp.float32)
    logits_ref[...] = logits

    row_max = jnp.max(logits, axis=1, keepdims=True)
    v = logits - row_max
    ex = jnp.exp(v)
    sum_ex = jnp.sum(ex, axis=1, keepdims=True)
    probs_ref[...] = ex / sum_ex

    n_experts = logits.shape[1]
    lane = jax.lax.broadcasted_iota(jnp.int32, logits.shape, 1)
    scale = jnp.float32(2097152.0)
    offset = jnp.int32(31457280)
    p = (jnp.maximum(v, jnp.float32(-14.9)) * scale).astype(jnp.int32)
    keys = jax.lax.bitcast_convert_type(
        ((p + offset) << 6) | (jnp.int32(n_experts - 1) - lane), jnp.float32
    )
    idxs = []
    qs = []
    for _ in range(_TOP_K):
        m = jnp.max(keys, axis=1, keepdims=True)
        m_bits = jax.lax.bitcast_convert_type(m, jnp.int32)
        idxs.append(jnp.int32(n_experts - 1) - (m_bits & jnp.int32(n_experts - 1)))
        qs.append((m_bits >> 6) - offset)
        keys = jnp.where(keys == m, jnp.float32(0.0), keys)

    top_v = jnp.concatenate(qs, axis=1).astype(jnp.float32) * (
        jnp.float32(1.0) / scale
    )
    top_ex = jnp.exp(top_v)
    tw_ref[...] = top_ex / jnp.sum(top_ex, axis=1, keepdims=True)
    idx_ref[...] = jnp.concatenate(idxs, axis=1)


@jax.jit
def kernel(x_flat, W_gate):
    n_tokens, d_model = x_flat.shape
    n_experts = W_gate.shape[1]
    grid = (n_tokens // _BLOCK_ROWS,)
    out_shapes = (
        jax.ShapeDtypeStruct((n_tokens, _TOP_K), jnp.int32),
        jax.ShapeDtypeStruct((n_tokens, _TOP_K), jnp.float32),
        jax.ShapeDtypeStruct((n_tokens, n_experts), jnp.float32),
        jax.ShapeDtypeStruct((n_tokens, n_experts), jnp.float32),
    )
    in_specs = [
        pl.BlockSpec((_BLOCK_ROWS, d_model), lambda i: (i, 0)),
        pl.BlockSpec((d_model, n_experts), lambda i: (0, 0)),
    ]
    out_specs = (
        pl.BlockSpec((_BLOCK_ROWS, _TOP_K), lambda i: (i, 0)),
        pl.BlockSpec((_BLOCK_ROWS, _TOP_K), lambda i: (i, 0)),
        pl.BlockSpec((_BLOCK_ROWS, n_experts), lambda i: (i, 0)),
        pl.BlockSpec((_BLOCK_ROWS, n_experts), lambda i: (i, 0)),
    )
    return pl.pallas_call(
        _router_block,
        grid=grid,
        in_specs=in_specs,
        out_specs=out_specs,
        out_shape=out_shapes,
    )(x_flat, W_gate)


# speedup vs baseline: 1.1232x; 1.1232x over previous
import jax
import jax.numpy as jnp
from jax.experimental import pallas as pl

_TOP_K = 8
_BLOCK_ROWS = 1024


def _router_block(x_ref, w_ref, idx_ref, tw_ref, probs_ref, logits_ref):
    logits = jnp.dot(x_ref[...], w_ref[...], preferred_element_type=jnp.float32)
    logits_ref[...] = logits

    row_max = jnp.max(logits, axis=1, keepdims=True)
    v = logits - row_max
    ex = jnp.exp(v)
    sum_ex = jnp.sum(ex, axis=1, keepdims=True)
    probs_ref[...] = ex / sum_ex

    n_experts = logits.shape[1]
    lane = jax.lax.broadcasted_iota(jnp.int32, logits.shape, 1)
    scale = jnp.float32(2097152.0)
    offset = jnp.int32(31457280)
    p = (jnp.maximum(v, jnp.float32(-14.9)) * scale).astype(jnp.int32)
    keys = jax.lax.bitcast_convert_type(
        ((p + offset) << 6) | (jnp.int32(n_experts - 1) - lane), jnp.float32
    )
    idxs = []
    qs = []
    for _ in range(_TOP_K):
        m = jnp.max(keys, axis=1, keepdims=True)
        m_bits = jax.lax.bitcast_convert_type(m, jnp.int32)
        idxs.append(jnp.int32(n_experts - 1) - (m_bits & jnp.int32(n_experts - 1)))
        qs.append((m_bits >> 6) - offset)
        keys = jnp.where(keys == m, jnp.float32(0.0), keys)

    top_v = jnp.concatenate(qs, axis=1).astype(jnp.float32) * (
        jnp.float32(1.0) / scale
    )
    top_ex = jnp.exp(top_v)
    tw_ref[...] = top_ex / jnp.sum(top_ex, axis=1, keepdims=True)
    idx_ref[...] = jnp.concatenate(idxs, axis=1)


@jax.jit
def kernel(x_flat, W_gate):
    n_tokens, d_model = x_flat.shape
    n_experts = W_gate.shape[1]
    grid = (n_tokens // _BLOCK_ROWS,)
    out_shapes = (
        jax.ShapeDtypeStruct((n_tokens, _TOP_K), jnp.int32),
        jax.ShapeDtypeStruct((n_tokens, _TOP_K), jnp.float32),
        jax.ShapeDtypeStruct((n_tokens, n_experts), jnp.float32),
        jax.ShapeDtypeStruct((n_tokens, n_experts), jnp.float32),
    )
    in_specs = [
        pl.BlockSpec((_BLOCK_ROWS, d_model), lambda i: (i, 0)),
        pl.BlockSpec((d_model, n_experts), lambda i: (0, 0)),
    ]
    out_specs = (
        pl.BlockSpec((_BLOCK_ROWS, _TOP_K), lambda i: (i, 0)),
        pl.BlockSpec((_BLOCK_ROWS, _TOP_K), lambda i: (i, 0)),
        pl.BlockSpec((_BLOCK_ROWS, n_experts), lambda i: (i, 0)),
        pl.BlockSpec((_BLOCK_ROWS, n_experts), lambda i: (i, 0)),
    )
    return pl.pallas_call(
        _router_block,
        grid=grid,
        in_specs=in_specs,
        out_specs=out_specs,
        out_shape=out_shapes,
    )(x_flat, W_gate)


# speedup vs baseline: 1.1392x; 1.0142x over previous
import jax
import jax.numpy as jnp
from jax.experimental import pallas as pl

_TOP_K = 8
_BLOCK_ROWS = 2048


def _router_block(x_ref, w_ref, idx_ref, tw_ref, probs_ref, logits_ref):
    logits = jnp.dot(x_ref[...], w_ref[...], preferred_element_type=jnp.float32)
    logits_ref[...] = logits

    row_max = jnp.max(logits, axis=1, keepdims=True)
    v = logits - row_max
    ex = jnp.exp(v)
    sum_ex = jnp.sum(ex, axis=1, keepdims=True)
    probs_ref[...] = ex / sum_ex

    n_experts = logits.shape[1]
    lane = jax.lax.broadcasted_iota(jnp.int32, logits.shape, 1)
    scale = jnp.float32(2097152.0)
    offset = jnp.int32(31457280)
    p = (jnp.maximum(v, jnp.float32(-14.9)) * scale).astype(jnp.int32)
    keys = jax.lax.bitcast_convert_type(
        ((p + offset) << 6) | (jnp.int32(n_experts - 1) - lane), jnp.float32
    )
    idxs = []
    qs = []
    for _ in range(_TOP_K):
        m = jnp.max(keys, axis=1, keepdims=True)
        m_bits = jax.lax.bitcast_convert_type(m, jnp.int32)
        idxs.append(jnp.int32(n_experts - 1) - (m_bits & jnp.int32(n_experts - 1)))
        qs.append((m_bits >> 6) - offset)
        keys = jnp.where(keys == m, jnp.float32(0.0), keys)

    top_v = jnp.concatenate(qs, axis=1).astype(jnp.float32) * (
        jnp.float32(1.0) / scale
    )
    top_ex = jnp.exp(top_v)
    tw_ref[...] = top_ex / jnp.sum(top_ex, axis=1, keepdims=True)
    idx_ref[...] = jnp.concatenate(idxs, axis=1)


@jax.jit
def kernel(x_flat, W_gate):
    n_tokens, d_model = x_flat.shape
    n_experts = W_gate.shape[1]
    grid = (n_tokens // _BLOCK_ROWS,)
    out_shapes = (
        jax.ShapeDtypeStruct((n_tokens, _TOP_K), jnp.int32),
        jax.ShapeDtypeStruct((n_tokens, _TOP_K), jnp.float32),
        jax.ShapeDtypeStruct((n_tokens, n_experts), jnp.float32),
        jax.ShapeDtypeStruct((n_tokens, n_experts), jnp.float32),
    )
    in_specs = [
        pl.BlockSpec((_BLOCK_ROWS, d_model), lambda i: (i, 0)),
        pl.BlockSpec((d_model, n_experts), lambda i: (0, 0)),
    ]
    out_specs = (
        pl.BlockSpec((_BLOCK_ROWS, _TOP_K), lambda i: (i, 0)),
        pl.BlockSpec((_BLOCK_ROWS, _TOP_K), lambda i: (i, 0)),
        pl.BlockSpec((_BLOCK_ROWS, n_experts), lambda i: (i, 0)),
        pl.BlockSpec((_BLOCK_ROWS, n_experts), lambda i: (i, 0)),
    )
    return pl.pallas_call(
        _router_block,
        grid=grid,
        in_specs=in_specs,
        out_specs=out_specs,
        out_shape=out_shapes,
    )(x_flat, W_gate)
